# Initial kernel scaffold; baseline (speedup 1.0000x reference)
import jax
import jax.numpy as jnp
from jax.experimental import pallas as pl


def kernel(feature1, feature2, cross_embedding):
    raise NotImplementedError("write your pallas kernel here")


# speedup vs baseline: 1.5051x; 1.5051x over previous
import functools

import jax
import jax.numpy as jnp
from jax import lax
from jax.experimental import pallas as pl
from jax.experimental.pallas import tpu as pltpu
from jax.experimental.pallas import tpu_sc as plsc

_NUM_BINS = 100000
_EMBED_DIM = 128
_BATCH = 16384

_NC = 2
_NS = 16
_L = 16
_NW = _NC * _NS
_B_PER_W = _BATCH // _NW
_CHUNK = 128
_N_CHUNKS = _B_PER_W // _CHUNK


def _hash_chunk(u1, u2):
    x = u1 * jnp.uint32(2654435761) + u2 * jnp.uint32(40503)
    x = x ^ (x >> jnp.uint32(16))
    x = x * jnp.uint32(2246822519)
    x = x ^ (x >> jnp.uint32(13))
    x = x * jnp.uint32(3266489917)
    x = x ^ (x >> jnp.uint32(16))
    return x % jnp.uint32(_NUM_BINS)


@functools.partial(
    pl.kernel,
    out_type=jax.ShapeDtypeStruct((_BATCH, _EMBED_DIM), jnp.float32),
    mesh=plsc.VectorSubcoreMesh(core_axis_name="c", subcore_axis_name="s"),
    scratch_types=[
        pltpu.VMEM((_B_PER_W,), jnp.int32),
        pltpu.VMEM((_B_PER_W,), jnp.int32),
        pltpu.VMEM((_B_PER_W,), jnp.int32),
        pltpu.VMEM((_B_PER_W, _EMBED_DIM), jnp.float32),
        pltpu.SemaphoreType.DMA,
    ],
)
def _sc_cross_lookup(f1_hbm, f2_hbm, table_hbm, out_hbm,
                     f1_v, f2_v, idx_v, rows_v, sem):
    wid = lax.axis_index("s") * _NC + lax.axis_index("c")
    base = wid * _B_PER_W

    pltpu.sync_copy(f1_hbm.at[pl.ds(base, _B_PER_W)], f1_v)
    pltpu.sync_copy(f2_hbm.at[pl.ds(base, _B_PER_W)], f2_v)

    for i in range(_B_PER_W // _L):
        sl = pl.ds(i * _L, _L)
        u1 = lax.bitcast_convert_type(f1_v[sl], jnp.uint32)
        u2 = lax.bitcast_convert_type(f2_v[sl], jnp.uint32)
        idx_v[sl] = lax.bitcast_convert_type(_hash_chunk(u1, u2), jnp.int32)

    copies = []
    for j in range(_N_CHUNKS):
        copies.append(
            pltpu.async_copy(
                table_hbm.at[idx_v.at[pl.ds(j * _CHUNK, _CHUNK)]],
                rows_v.at[pl.ds(j * _CHUNK, _CHUNK)],
                sem,
            )
        )
    for c in copies:
        c.wait()

    pltpu.sync_copy(rows_v, out_hbm.at[pl.ds(base, _B_PER_W)])


def kernel(feature1, feature2, cross_embedding):
    return _sc_cross_lookup(feature1, feature2, cross_embedding)


# speedup vs baseline: 1.5360x; 1.0205x over previous
import functools

import jax
import jax.numpy as jnp
from jax import lax
from jax.experimental import pallas as pl
from jax.experimental.pallas import tpu as pltpu
from jax.experimental.pallas import tpu_sc as plsc

_NUM_BINS = 100000
_EMBED_DIM = 128
_BATCH = 16384

_NC = 2
_NS = 16
_L = 16
_NW = _NC * _NS
_B_PER_W = _BATCH // _NW
_CHUNK = 128
_N_CHUNKS = _B_PER_W // _CHUNK


def _hash_chunk(u1, u2):
    x = u1 * jnp.uint32(2654435761) + u2 * jnp.uint32(40503)
    x = x ^ (x >> jnp.uint32(16))
    x = x * jnp.uint32(2246822519)
    x = x ^ (x >> jnp.uint32(13))
    x = x * jnp.uint32(3266489917)
    x = x ^ (x >> jnp.uint32(16))
    return x % jnp.uint32(_NUM_BINS)


@functools.partial(
    pl.kernel,
    out_type=jax.ShapeDtypeStruct((_BATCH, _EMBED_DIM), jnp.float32),
    mesh=plsc.VectorSubcoreMesh(core_axis_name="c", subcore_axis_name="s"),
    scratch_types=[
        pltpu.VMEM((_B_PER_W,), jnp.int32),
        pltpu.VMEM((_B_PER_W,), jnp.int32),
        pltpu.VMEM((_B_PER_W,), jnp.int32),
        pltpu.VMEM((_B_PER_W, _EMBED_DIM), jnp.float32),
        pltpu.SemaphoreType.DMA,
        pltpu.SemaphoreType.DMA((_N_CHUNKS,)),
        pltpu.SemaphoreType.DMA,
    ],
)
def _sc_cross_lookup(f1_hbm, f2_hbm, table_hbm, out_hbm,
                     f1_v, f2_v, idx_v, rows_v, fsem, gsem, ssem):
    wid = lax.axis_index("s") * _NC + lax.axis_index("c")
    base = wid * _B_PER_W

    cf1 = pltpu.async_copy(f1_hbm.at[pl.ds(base, _B_PER_W)], f1_v, fsem)
    cf2 = pltpu.async_copy(f2_hbm.at[pl.ds(base, _B_PER_W)], f2_v, fsem)
    cf1.wait()
    cf2.wait()

    gathers = []
    for j in range(_N_CHUNKS):
        for i in range(j * (_CHUNK // _L), (j + 1) * (_CHUNK // _L)):
            sl = pl.ds(i * _L, _L)
            u1 = lax.bitcast_convert_type(f1_v[sl], jnp.uint32)
            u2 = lax.bitcast_convert_type(f2_v[sl], jnp.uint32)
            idx_v[sl] = lax.bitcast_convert_type(_hash_chunk(u1, u2), jnp.int32)
        gathers.append(
            pltpu.async_copy(
                table_hbm.at[idx_v.at[pl.ds(j * _CHUNK, _CHUNK)]],
                rows_v.at[pl.ds(j * _CHUNK, _CHUNK)],
                gsem.at[j],
            )
        )

    stores = []
    for j in range(_N_CHUNKS):
        gathers[j].wait()
        stores.append(
            pltpu.async_copy(
                rows_v.at[pl.ds(j * _CHUNK, _CHUNK)],
                out_hbm.at[pl.ds(base + j * _CHUNK, _CHUNK)],
                ssem,
            )
        )
    for s in stores:
        s.wait()


def kernel(feature1, feature2, cross_embedding):
    return _sc_cross_lookup(feature1, feature2, cross_embedding)
